# baseline (speedup 1.0000x reference)
import functools

import jax
import jax.numpy as jnp
from jax.experimental import pallas as pl
from jax.experimental.pallas import tpu as pltpu

_BINS = 32
_SIGMA = 0.05
_LOG2E = 1.4426950408889634
_C2 = -0.5 * _LOG2E / (_SIGMA * _SIGMA)

_BR = 512


def _hist_kernel(br, w, xa_ref, xb_ref, bins2_ref, o_ref):
    k = pl.program_id(1)
    bins2_col = bins2_ref[:, 0:1]

    def pair_hist(rp):
        r = 2 * rp
        pa = pltpu.pack_elementwise(
            [xa_ref[0, r:r + 1, :], xa_ref[0, r + 1:r + 2, :]],
            packed_dtype=jnp.bfloat16)
        pb = pltpu.pack_elementwise(
            [xb_ref[0, r:r + 1, :], xb_ref[0, r + 1:r + 2, :]],
            packed_dtype=jnp.bfloat16)
        pa_f = pltpu.bitcast(pa, jnp.float32)
        pb_f = pltpu.bitcast(pb, jnp.float32)
        xa2 = pltpu.bitcast(
            jnp.broadcast_to(pa_f, (_BINS, w)), jnp.bfloat16)
        xb2 = pltpu.bitcast(
            jnp.broadcast_to(pb_f, (_BINS, w)), jnp.bfloat16)
        da = xa2 - bins2_col
        db = xb2 - bins2_col
        ka = jnp.exp2(da * da * _C2)
        kb = jnp.exp2(db * db * _C2)
        return jax.lax.dot_general(
            ka, kb, (((1,), (1,)), ((), ())),
            preferred_element_type=jnp.float32)

    nacc = 4
    accs = [pair_hist(j) for j in range(nacc)]
    for rp in range(nacc, br // 2):
        j = rp % nacc
        accs[j] = accs[j] + pair_hist(rp)
    h = accs[0]
    for j in range(1, nacc):
        h = h + accs[j]

    @pl.when(k == 0)
    def _():
        o_ref[0] = h

    @pl.when(k != 0)
    def _():
        o_ref[0] = o_ref[0] + h


def kernel(image):
    n, c, h, w = image.shape
    x = image.reshape(n * c, h, w)
    bins2 = jnp.broadcast_to(
        jnp.repeat(jnp.linspace(0.0, 1.0, _BINS, dtype=jnp.float32), 2
                   ).astype(jnp.bfloat16)[:, None],
        (2 * _BINS, 128))
    br = min(_BR, h)
    num_k = h // br
    out = pl.pallas_call(
        functools.partial(_hist_kernel, br, w),
        grid=(n, num_k),
        in_specs=[
            pl.BlockSpec((1, br, w), lambda i, k: (2 * i, k, 0)),
            pl.BlockSpec((1, br, w), lambda i, k: (2 * i + 1, k, 0)),
            pl.BlockSpec((2 * _BINS, 128), lambda i, k: (0, 0)),
        ],
        out_specs=pl.BlockSpec((1, 2 * _BINS, 2 * _BINS),
                               lambda i, k: (i, 0, 0)),
        out_shape=jax.ShapeDtypeStruct((n, 2 * _BINS, 2 * _BINS),
                                       jnp.float32),
        compiler_params=pltpu.CompilerParams(
            dimension_semantics=("parallel", "arbitrary")),
    )(x, x, bins2)
    hist = out[:, 0::2, 0::2] + out[:, 1::2, 1::2]
    return hist[:, None, :, :]


# speedup vs baseline: 1.0005x; 1.0005x over previous
import functools

import jax
import jax.numpy as jnp
from jax.experimental import pallas as pl
from jax.experimental.pallas import tpu as pltpu

_BINS = 32
_SIGMA = 0.05
_LOG2E = 1.4426950408889634
_C2 = -0.5 * _LOG2E / (_SIGMA * _SIGMA)

_BR = 512


def _hist_kernel(br, w, xa_ref, xb_ref, bins2_ref, o_ref):
    k = pl.program_id(1)
    bins2_col = bins2_ref[:, 0:1]

    def pair_hist(rp):
        r = 2 * rp
        pa = pltpu.pack_elementwise(
            [xa_ref[0, r:r + 1, :], xa_ref[0, r + 1:r + 2, :]],
            packed_dtype=jnp.bfloat16)
        pb = pltpu.pack_elementwise(
            [xb_ref[0, r:r + 1, :], xb_ref[0, r + 1:r + 2, :]],
            packed_dtype=jnp.bfloat16)
        pa_f = pltpu.bitcast(pa, jnp.float32)
        pb_f = pltpu.bitcast(pb, jnp.float32)
        xa2 = pltpu.bitcast(
            jnp.broadcast_to(pa_f, (_BINS, w)), jnp.bfloat16)
        xb2 = pltpu.bitcast(
            jnp.broadcast_to(pb_f, (_BINS, w)), jnp.bfloat16)
        da = xa2 - bins2_col
        db = xb2 - bins2_col
        ka = jnp.exp2(da * da * _C2)
        kb = jnp.exp2(db * db * _C2)
        kb_t = kb.T
        return jax.lax.dot_general(
            ka, kb_t, (((1,), (0,)), ((), ())),
            preferred_element_type=jnp.float32)

    nacc = 4
    accs = [pair_hist(j) for j in range(nacc)]
    for rp in range(nacc, br // 2):
        j = rp % nacc
        accs[j] = accs[j] + pair_hist(rp)
    h = accs[0]
    for j in range(1, nacc):
        h = h + accs[j]

    @pl.when(k == 0)
    def _():
        o_ref[0] = h

    @pl.when(k != 0)
    def _():
        o_ref[0] = o_ref[0] + h


def kernel(image):
    n, c, h, w = image.shape
    x = image.reshape(n * c, h, w)
    bins2 = jnp.broadcast_to(
        jnp.repeat(jnp.linspace(0.0, 1.0, _BINS, dtype=jnp.float32), 2
                   ).astype(jnp.bfloat16)[:, None],
        (2 * _BINS, 128))
    br = min(_BR, h)
    num_k = h // br
    out = pl.pallas_call(
        functools.partial(_hist_kernel, br, w),
        grid=(n, num_k),
        in_specs=[
            pl.BlockSpec((1, br, w), lambda i, k: (2 * i, k, 0)),
            pl.BlockSpec((1, br, w), lambda i, k: (2 * i + 1, k, 0)),
            pl.BlockSpec((2 * _BINS, 128), lambda i, k: (0, 0)),
        ],
        out_specs=pl.BlockSpec((1, 2 * _BINS, 2 * _BINS),
                               lambda i, k: (i, 0, 0)),
        out_shape=jax.ShapeDtypeStruct((n, 2 * _BINS, 2 * _BINS),
                                       jnp.float32),
        compiler_params=pltpu.CompilerParams(
            dimension_semantics=("parallel", "arbitrary")),
    )(x, x, bins2)
    hist = out[:, 0::2, 0::2] + out[:, 1::2, 1::2]
    return hist[:, None, :, :]


# speedup vs baseline: 1.0280x; 1.0275x over previous
import functools

import jax
import jax.numpy as jnp
from jax.experimental import pallas as pl
from jax.experimental.pallas import tpu as pltpu

_BINS = 32
_SIGMA = 0.05
_LOG2E = 1.4426950408889634
_C2 = -0.5 * _LOG2E / (_SIGMA * _SIGMA)

_BR = 512


def _hist_kernel(br, w, xa_ref, xb_ref, bins2_ref, o_ref):
    k = pl.program_id(1)
    bins2_col = bins2_ref[:, 0:1]

    def pair_hist(rp):
        r = 2 * rp
        pa = pltpu.pack_elementwise(
            [xa_ref[0, r:r + 1, :], xa_ref[0, r + 1:r + 2, :]],
            packed_dtype=jnp.bfloat16)
        pb = pltpu.pack_elementwise(
            [xb_ref[0, r:r + 1, :], xb_ref[0, r + 1:r + 2, :]],
            packed_dtype=jnp.bfloat16)
        pa_f = pltpu.bitcast(pa, jnp.float32)
        pb_f = pltpu.bitcast(pb, jnp.float32)
        xa2 = pltpu.bitcast(
            jnp.broadcast_to(pa_f, (_BINS, w)), jnp.bfloat16)
        xb2 = pltpu.bitcast(
            jnp.broadcast_to(pb_f, (_BINS, w)), jnp.bfloat16)
        da = xa2 - bins2_col
        db = xb2 - bins2_col
        ka = jnp.exp2(da * da * _C2).astype(jnp.float8_e4m3fn)
        kb = jnp.exp2(db * db * _C2).astype(jnp.float8_e4m3fn)
        return jax.lax.dot_general(
            ka, kb, (((1,), (1,)), ((), ())),
            preferred_element_type=jnp.float32)

    nacc = 4
    accs = [pair_hist(j) for j in range(nacc)]
    for rp in range(nacc, br // 2):
        j = rp % nacc
        accs[j] = accs[j] + pair_hist(rp)
    h = accs[0]
    for j in range(1, nacc):
        h = h + accs[j]

    @pl.when(k == 0)
    def _():
        o_ref[0] = h

    @pl.when(k != 0)
    def _():
        o_ref[0] = o_ref[0] + h


def kernel(image):
    n, c, h, w = image.shape
    x = image.reshape(n * c, h, w)
    bins2 = jnp.broadcast_to(
        jnp.repeat(jnp.linspace(0.0, 1.0, _BINS, dtype=jnp.float32), 2
                   ).astype(jnp.bfloat16)[:, None],
        (2 * _BINS, 128))
    br = min(_BR, h)
    num_k = h // br
    out = pl.pallas_call(
        functools.partial(_hist_kernel, br, w),
        grid=(n, num_k),
        in_specs=[
            pl.BlockSpec((1, br, w), lambda i, k: (2 * i, k, 0)),
            pl.BlockSpec((1, br, w), lambda i, k: (2 * i + 1, k, 0)),
            pl.BlockSpec((2 * _BINS, 128), lambda i, k: (0, 0)),
        ],
        out_specs=pl.BlockSpec((1, 2 * _BINS, 2 * _BINS),
                               lambda i, k: (i, 0, 0)),
        out_shape=jax.ShapeDtypeStruct((n, 2 * _BINS, 2 * _BINS),
                                       jnp.float32),
        compiler_params=pltpu.CompilerParams(
            dimension_semantics=("parallel", "arbitrary")),
    )(x, x, bins2)
    hist = out[:, 0::2, 0::2] + out[:, 1::2, 1::2]
    return hist[:, None, :, :]


# speedup vs baseline: 1.0289x; 1.0008x over previous
import functools

import jax
import jax.numpy as jnp
from jax.experimental import pallas as pl
from jax.experimental.pallas import tpu as pltpu

_BINS = 32
_SIGMA = 0.05
_LOG2E = 1.4426950408889634
_C2 = -0.5 * _LOG2E / (_SIGMA * _SIGMA)

_BR = 512


def _hist_kernel(br, w, xa_ref, xb_ref, bins2_ref, o_ref):
    k = pl.program_id(1)
    bins2_col = bins2_ref[:, 0:1]

    def pair_k(x_ref, rp):
        r = 2 * rp
        p = pltpu.pack_elementwise(
            [x_ref[0, r:r + 1, :], x_ref[0, r + 1:r + 2, :]],
            packed_dtype=jnp.bfloat16)
        p_f = pltpu.bitcast(p, jnp.float32)
        x2 = pltpu.bitcast(
            jnp.broadcast_to(p_f, (_BINS, w)), jnp.bfloat16)
        d = x2 - bins2_col
        return jnp.exp2(d * d * _C2).astype(jnp.float8_e4m3fn)

    grp = 4

    def group_hist(g):
        kas = [pair_k(xa_ref, g * grp + j) for j in range(grp)]
        kbs = [pair_k(xb_ref, g * grp + j) for j in range(grp)]
        ka = jnp.concatenate(kas, axis=1)
        kb = jnp.concatenate(kbs, axis=1)
        return jax.lax.dot_general(
            ka, kb, (((1,), (1,)), ((), ())),
            preferred_element_type=jnp.float32)

    ngrp = br // 2 // grp
    h = group_hist(0)
    for g in range(1, ngrp):
        h = h + group_hist(g)

    @pl.when(k == 0)
    def _():
        o_ref[0] = h

    @pl.when(k != 0)
    def _():
        o_ref[0] = o_ref[0] + h


def kernel(image):
    n, c, h, w = image.shape
    x = image.reshape(n * c, h, w)
    bins2 = jnp.broadcast_to(
        jnp.repeat(jnp.linspace(0.0, 1.0, _BINS, dtype=jnp.float32), 2
                   ).astype(jnp.bfloat16)[:, None],
        (2 * _BINS, 128))
    br = min(_BR, h)
    num_k = h // br
    out = pl.pallas_call(
        functools.partial(_hist_kernel, br, w),
        grid=(n, num_k),
        in_specs=[
            pl.BlockSpec((1, br, w), lambda i, k: (2 * i, k, 0)),
            pl.BlockSpec((1, br, w), lambda i, k: (2 * i + 1, k, 0)),
            pl.BlockSpec((2 * _BINS, 128), lambda i, k: (0, 0)),
        ],
        out_specs=pl.BlockSpec((1, 2 * _BINS, 2 * _BINS),
                               lambda i, k: (i, 0, 0)),
        out_shape=jax.ShapeDtypeStruct((n, 2 * _BINS, 2 * _BINS),
                                       jnp.float32),
        compiler_params=pltpu.CompilerParams(
            dimension_semantics=("parallel", "arbitrary")),
    )(x, x, bins2)
    hist = out[:, 0::2, 0::2] + out[:, 1::2, 1::2]
    return hist[:, None, :, :]


# speedup vs baseline: 1.0624x; 1.0325x over previous
import functools

import jax
import jax.numpy as jnp
from jax.experimental import pallas as pl
from jax.experimental.pallas import tpu as pltpu

_BINS = 32
_SIGMA = 0.05
_LOG2E = 1.4426950408889634
_C2 = -0.5 * _LOG2E / (_SIGMA * _SIGMA)
_S = (0.5 * _LOG2E) ** 0.5 / _SIGMA

_BR = 512


def _hist_kernel(br, xa_ref, xb_ref, bins_ref, o_ref):
    k = pl.program_id(1)
    bins_col = bins_ref[:, 0:1]

    def row_hist(r):
        xa = xa_ref[0, r:r + 1, :] * _S
        xb = xb_ref[0, r:r + 1, :] * _S
        da = (xa - bins_col).astype(jnp.bfloat16)
        db = (xb - bins_col).astype(jnp.bfloat16)
        ka = jnp.exp2(-(da * da))
        kb = jnp.exp2(-(db * db))
        return jax.lax.dot_general(
            ka, kb, (((1,), (1,)), ((), ())),
            preferred_element_type=jnp.float32)

    h = row_hist(0)
    for r in range(1, br):
        h = h + row_hist(r)

    @pl.when(k == 0)
    def _():
        o_ref[0] = h

    @pl.when(k != 0)
    def _():
        o_ref[0] = o_ref[0] + h


def kernel(image):
    n, c, h, w = image.shape
    x = image.reshape(n * c, h, w)
    bins = jnp.broadcast_to(
        (jnp.linspace(0.0, 1.0, _BINS, dtype=jnp.float32) * _S)[:, None],
        (_BINS, 128))
    br = min(_BR, h)
    num_k = h // br
    out = pl.pallas_call(
        functools.partial(_hist_kernel, br),
        grid=(n, num_k),
        in_specs=[
            pl.BlockSpec((1, br, w), lambda i, k: (2 * i, k, 0)),
            pl.BlockSpec((1, br, w), lambda i, k: (2 * i + 1, k, 0)),
            pl.BlockSpec((_BINS, 128), lambda i, k: (0, 0)),
        ],
        out_specs=pl.BlockSpec((1, _BINS, _BINS), lambda i, k: (i, 0, 0)),
        out_shape=jax.ShapeDtypeStruct((n, _BINS, _BINS), jnp.float32),
        compiler_params=pltpu.CompilerParams(
            dimension_semantics=("parallel", "arbitrary")),
    )(x, x, bins)
    return out[:, None, :, :]
